# Initial kernel scaffold; baseline (speedup 1.0000x reference)
import jax
import jax.numpy as jnp
from jax.experimental import pallas as pl


def kernel(x, gate_w, w1, b1, w2, b2):
    raise NotImplementedError("write your pallas kernel here")


# speedup vs baseline: 2.4643x; 2.4643x over previous
import functools

import jax
import jax.numpy as jnp
from jax import lax
from jax.experimental import pallas as pl
from jax.experimental.pallas import tpu as pltpu

T = 2048
D = 768
F = 3072
E = 8
K = 2
BLK = 256
NP = K * T
P_MAX = NP + E * BLK - BLK * 2
P_MAX = 6144
NT = P_MAX // BLK
NCHUNK = 12
CH = P_MAX // NCHUNK


def _router_kernel(x_ref, gw_ref, dest_ref, w2k_ref, sortv_ref, meta_ref,
                   loss_ref):
    x = x_ref[...]
    gw = gw_ref[...]
    logits = lax.dot_general(x, gw, (((1,), (1,)), ((), ())),
                             preferred_element_type=jnp.float32)
    m = jnp.max(logits, axis=1, keepdims=True)
    ex = jnp.exp(logits - m)
    probs = ex / jnp.sum(ex, axis=1, keepdims=True)

    mean_p = jnp.mean(probs, axis=0, keepdims=True)
    loss_ref[...] = (float(E) * jnp.sum(mean_p * mean_p)).reshape(1, 1)

    iota_e = lax.broadcasted_iota(jnp.int32, (T, E), 1).astype(jnp.float32)
    m1 = jnp.max(probs, axis=1, keepdims=True)
    i1 = jnp.min(jnp.where(probs >= m1, iota_e, float(E)), axis=1,
                 keepdims=True)
    h0 = (iota_e == i1).astype(jnp.float32)
    probs_m = jnp.where(iota_e == i1, -1.0, probs)
    m2 = jnp.max(probs_m, axis=1, keepdims=True)
    i2 = jnp.min(jnp.where(probs_m >= m2, iota_e, float(E)), axis=1,
                 keepdims=True)
    h1 = (iota_e == i2).astype(jnp.float32)

    denom = m1 + m2
    w0 = m1 / denom
    w1_ = m2 / denom

    ct = h0 + h1
    s = 1
    while s < T:
        shifted = jnp.concatenate(
            [jnp.zeros((s, E), jnp.float32), ct[: T - s, :]], axis=0)
        ct = ct + shifted
        s *= 2
    cnt = ct[T - 1:T, :]
    pc = jnp.ceil(cnt * (1.0 / BLK)) * float(BLK)

    bi = lax.broadcasted_iota(jnp.int32, (E, E), 0).astype(jnp.float32)
    ei = lax.broadcasted_iota(jnp.int32, (E, E), 1).astype(jnp.float32)
    u_excl = (bi < ei).astype(jnp.float32)
    u_incl = (bi >= ei).astype(jnp.float32)
    bo = lax.dot_general(pc, u_excl, (((1,), (0,)), ((), ())),
                         preferred_element_type=jnp.float32,
                         precision=lax.Precision.HIGHEST)
    boincl_col = lax.dot_general(u_incl, pc, (((1,), (1,)), ((), ())),
                                 preferred_element_type=jnp.float32,
                                 precision=lax.Precision.HIGHEST)

    d0 = jnp.sum(h0 * (bo + ct - 1.0), axis=1, keepdims=True)
    d1 = jnp.sum(h1 * (bo + ct - 1.0), axis=1, keepdims=True)
    dest_ref[...] = jnp.concatenate([d0, d1], axis=1).astype(jnp.int32)
    w2k_ref[...] = jnp.concatenate([w0, w1_], axis=1)

    ti = lax.broadcasted_iota(jnp.int32, (E, 2 * NT), 1).astype(jnp.float32) * float(BLK)
    raw = jnp.sum((ti >= boincl_col).astype(jnp.float32), axis=0,
                  keepdims=True)
    e_iota_row = lax.broadcasted_iota(jnp.int32, (1, E), 1).astype(jnp.float32)
    me = jnp.max(jnp.where(cnt > 0.0, e_iota_row, -1.0), axis=1,
                 keepdims=True)
    e_of_tile = jnp.minimum(raw, me)
    nt_used = jnp.sum(pc) * (1.0 / BLK)
    meta_ref[0:1, 0:2 * NT] = e_of_tile.astype(jnp.int32)
    meta_ref[0:1, 2 * NT:4 * NT] = jnp.full((1, 2 * NT), 0, jnp.int32) + (
        nt_used.astype(jnp.int32))

    tok_col = lax.broadcasted_iota(jnp.int32, (T, 1), 0).astype(jnp.float32)
    rhs0 = jnp.concatenate([tok_col, w0], axis=1)
    rhs1 = jnp.concatenate([tok_col, w1_], axis=1)
    for c in range(NCHUNK):
        jiota = lax.broadcasted_iota(jnp.int32, (T, CH), 1).astype(jnp.float32) + float(c * CH)
        cmp0 = (jiota == d0).astype(jnp.float32)
        cmp1 = (jiota == d1).astype(jnp.float32)
        chunk = lax.dot_general(cmp0, rhs0, (((0,), (0,)), ((), ())),
                                preferred_element_type=jnp.float32,
                                precision=lax.Precision.HIGHEST)
        chunk = chunk + lax.dot_general(cmp1, rhs1, (((0,), (0,)), ((), ())),
                                        preferred_element_type=jnp.float32,
                                        precision=lax.Precision.HIGHEST)
        sortv_ref[c * CH:(c + 1) * CH, :] = chunk


def _ffn_kernel(meta_ref, x_ref, w1_ref, b1_ref, w2_ref, b2_ref, tok_ref,
                ws_ref, out_ref):
    i = pl.program_id(0)
    nt_used = meta_ref[2 * NT]

    @pl.when(i == 0)
    def _init():
        out_ref[...] = jnp.zeros((T, D), jnp.float32)

    @pl.when(i < nt_used)
    def _body():
        tok = tok_ref[0, 0, :]
        ws = ws_ref[0, 0, :]
        tok_c = tok.reshape(BLK, 1)
        ti = lax.broadcasted_iota(jnp.int32, (BLK, T), 1)
        oh = (ti == tok_c).astype(jnp.bfloat16)
        xg = lax.dot_general(oh, x_ref[...], (((1,), (0,)), ((), ())),
                             preferred_element_type=jnp.float32)
        h = lax.dot_general(xg.astype(jnp.bfloat16), w1_ref[0],
                            (((1,), (0,)), ((), ())),
                            preferred_element_type=jnp.float32)
        h = h + b1_ref[0, 0, :][None, :]
        g = 0.5 * h * (1.0 + lax.erf(h * 0.7071067811865476))
        y = lax.dot_general(g.astype(jnp.bfloat16), w2_ref[0],
                            (((1,), (0,)), ((), ())),
                            preferred_element_type=jnp.float32)
        y = y + b2_ref[0, 0, :][None, :]
        wy = (y * ws.reshape(BLK, 1)).astype(jnp.bfloat16)
        tj = lax.broadcasted_iota(jnp.int32, (T, BLK), 0)
        oht = (tj == tok.reshape(1, BLK)).astype(jnp.bfloat16)
        out_ref[...] += lax.dot_general(oht, wy, (((1,), (0,)), ((), ())),
                                        preferred_element_type=jnp.float32)


def _router(xt, gate_w):
    return pl.pallas_call(
        _router_kernel,
        out_shape=[
            jax.ShapeDtypeStruct((T, K), jnp.int32),
            jax.ShapeDtypeStruct((T, K), jnp.float32),
            jax.ShapeDtypeStruct((P_MAX, 2), jnp.float32),
            jax.ShapeDtypeStruct((1, 4 * NT), jnp.int32),
            jax.ShapeDtypeStruct((1, 1), jnp.float32),
        ],
    )(xt, gate_w)


def _ffn(meta, x16, w1_16, b1r, w2_16, b2r, tok_s, ws_s):
    grid_spec = pltpu.PrefetchScalarGridSpec(
        num_scalar_prefetch=1,
        grid=(NT,),
        in_specs=[
            pl.BlockSpec((T, D), lambda i, m: (0, 0)),
            pl.BlockSpec((1, D, F), lambda i, m: (m[i], 0, 0)),
            pl.BlockSpec((1, 1, F), lambda i, m: (m[i], 0, 0)),
            pl.BlockSpec((1, F, D), lambda i, m: (m[i], 0, 0)),
            pl.BlockSpec((1, 1, D), lambda i, m: (m[i], 0, 0)),
            pl.BlockSpec((1, 1, BLK), lambda i, m: (i, 0, 0)),
            pl.BlockSpec((1, 1, BLK), lambda i, m: (i, 0, 0)),
        ],
        out_specs=pl.BlockSpec((T, D), lambda i, m: (0, 0)),
    )
    return pl.pallas_call(
        _ffn_kernel,
        grid_spec=grid_spec,
        out_shape=jax.ShapeDtypeStruct((T, D), jnp.float32),
    )(meta, x16, w1_16, b1r, w2_16, b2r, tok_s, ws_s)


def kernel(x, gate_w, w1, b1, w2, b2):
    orig_shape = x.shape
    xt = x.reshape(T, D)
    dest, w2k, sortv, meta, loss = _router(xt, gate_w)
    del dest, w2k
    tok_s = sortv[:, 0].astype(jnp.int32).reshape(NT, 1, BLK)
    ws_s = sortv[:, 1].reshape(NT, 1, BLK)
    out = _ffn(meta.reshape(4 * NT), xt.astype(jnp.bfloat16),
               w1.astype(jnp.bfloat16), b1.reshape(E, 1, F),
               w2.astype(jnp.bfloat16), b2.reshape(E, 1, D), tok_s, ws_s)
    return out.reshape(orig_shape), loss.reshape(())


# speedup vs baseline: 3.0421x; 1.2345x over previous
import functools

import jax
import jax.numpy as jnp
from jax import lax
from jax.experimental import pallas as pl
from jax.experimental.pallas import tpu as pltpu
from jax.experimental.pallas import tpu_sc as plsc

T = 2048
D = 768
F = 3072
E = 8
K = 2
BLK = 256
NP = K * T
P_MAX = 6144
NT = P_MAX // BLK
NW = 32
NPW = NP // NW
MW = 2 * NT


def _router_kernel(x_ref, gw_ref, dpack_ref, meta_ref, loss_ref):
    x = x_ref[...]
    gw = gw_ref[...]
    logits = lax.dot_general(x, gw, (((1,), (1,)), ((), ())),
                             preferred_element_type=jnp.float32)
    m = jnp.max(logits, axis=1, keepdims=True)
    ex = jnp.exp(logits - m)
    probs = ex / jnp.sum(ex, axis=1, keepdims=True)

    mean_p = jnp.mean(probs, axis=0, keepdims=True)
    loss_ref[...] = (float(E) * jnp.sum(mean_p * mean_p)).reshape(1, 1)

    iota_e = lax.broadcasted_iota(jnp.int32, (T, E), 1).astype(jnp.float32)
    m1 = jnp.max(probs, axis=1, keepdims=True)
    i1 = jnp.min(jnp.where(probs >= m1, iota_e, float(E)), axis=1,
                 keepdims=True)
    h0 = (iota_e == i1).astype(jnp.float32)
    probs_m = jnp.where(iota_e == i1, -1.0, probs)
    m2 = jnp.max(probs_m, axis=1, keepdims=True)
    i2 = jnp.min(jnp.where(probs_m >= m2, iota_e, float(E)), axis=1,
                 keepdims=True)
    h1 = (iota_e == i2).astype(jnp.float32)

    denom = m1 + m2
    w0 = m1 / denom
    w1_ = m2 / denom

    ct = h0 + h1
    s = 1
    while s < T:
        shifted = jnp.concatenate(
            [jnp.zeros((s, E), jnp.float32), ct[: T - s, :]], axis=0)
        ct = ct + shifted
        s *= 2
    cnt = ct[T - 1:T, :]
    pc = jnp.ceil(cnt * (1.0 / BLK)) * float(BLK)

    bi = lax.broadcasted_iota(jnp.int32, (E, E), 0).astype(jnp.float32)
    ei = lax.broadcasted_iota(jnp.int32, (E, E), 1).astype(jnp.float32)
    u_excl = (bi < ei).astype(jnp.float32)
    u_incl = (bi >= ei).astype(jnp.float32)
    ident = (bi == ei).astype(jnp.float32)
    bo = lax.dot_general(pc, u_excl, (((1,), (0,)), ((), ())),
                         preferred_element_type=jnp.float32,
                         precision=lax.Precision.HIGHEST)
    boincl_col = lax.dot_general(u_incl, pc, (((1,), (1,)), ((), ())),
                                 preferred_element_type=jnp.float32,
                                 precision=lax.Precision.HIGHEST)
    pc_col = lax.dot_general(ident, pc, (((1,), (1,)), ((), ())),
                             preferred_element_type=jnp.float32,
                             precision=lax.Precision.HIGHEST)
    cnt_col = lax.dot_general(ident, cnt, (((1,), (1,)), ((), ())),
                              preferred_element_type=jnp.float32,
                              precision=lax.Precision.HIGHEST)
    bo_col = boincl_col - pc_col

    d0 = jnp.sum(h0 * (bo + ct - 1.0), axis=1, keepdims=True)
    d1 = jnp.sum(h1 * (bo + ct - 1.0), axis=1, keepdims=True)
    li = lax.broadcasted_iota(jnp.int32, (T, 128), 1)
    dpack_ref[...] = (jnp.where(li == 0, d0, 0.0) +
                      jnp.where(li == 1, d1, 0.0) +
                      jnp.where(li == 2, w0, 0.0) +
                      jnp.where(li == 3, w1_, 0.0))

    ti = lax.broadcasted_iota(jnp.int32, (E, MW), 1).astype(
        jnp.float32) * float(BLK)
    raw = jnp.sum((ti >= boincl_col).astype(jnp.float32), axis=0,
                  keepdims=True)
    e_iota_row = lax.broadcasted_iota(jnp.int32, (1, E), 1).astype(jnp.float32)
    me = jnp.max(jnp.where(cnt > 0.0, e_iota_row, -1.0), axis=1,
                 keepdims=True)
    e_of_tile = jnp.minimum(raw, me)
    nt_used = jnp.sum(pc) * (1.0 / BLK)
    e_iota_col = lax.broadcasted_iota(jnp.int32, (E, 1), 0).astype(jnp.float32)
    tile_oh = (e_of_tile == e_iota_col).astype(jnp.float32)
    end_of_tile = jnp.sum(tile_oh * (bo_col + cnt_col), axis=0,
                          keepdims=True)
    meta = jnp.concatenate(
        [e_of_tile,
         jnp.zeros((1, MW), jnp.float32) + nt_used,
         end_of_tile], axis=1).astype(jnp.int32)
    meta_ref[...] = meta


@functools.lru_cache(maxsize=1)
def _make_sc_dispatch():
    mesh = plsc.VectorSubcoreMesh(core_axis_name="c", subcore_axis_name="s")

    @functools.partial(
        pl.kernel,
        mesh=mesh,
        out_type=[
            jax.ShapeDtypeStruct((P_MAX,), jnp.int32),
            jax.ShapeDtypeStruct((P_MAX,), jnp.float32),
        ],
        scratch_types=[
            pltpu.VMEM((NPW,), jnp.int32),
            pltpu.VMEM((NPW,), jnp.int32),
            pltpu.VMEM((NPW,), jnp.float32),
        ],
    )
    def sc_dispatch(dest_hbm, w_hbm, tok_out, w_out, idx_v, tokv, wv):
        wid = lax.axis_index("s") * 2 + lax.axis_index("c")
        base = wid * NPW
        pltpu.sync_copy(dest_hbm.at[pl.ds(base, NPW)], idx_v)
        pltpu.sync_copy(w_hbm.at[pl.ds(base, NPW)], wv)
        for j in range(NPW // 16):
            p = lax.iota(jnp.int32, 16) + (base + j * 16)
            tokv[pl.ds(j * 16, 16)] = lax.shift_right_logical(p, 1)
        pltpu.sync_copy(tokv, tok_out.at[idx_v])
        pltpu.sync_copy(wv, w_out.at[idx_v])

    return sc_dispatch


def _ffn_kernel(meta_ref, x_ref, w1_ref, b1_ref, w2_ref, b2_ref, tok_ref,
                ws_ref, out_ref):
    i = pl.program_id(0)
    nt_used = meta_ref[MW]

    @pl.when(i == 0)
    def _init():
        out_ref[...] = jnp.zeros((T, D), jnp.float32)

    @pl.when(i < nt_used)
    def _body():
        end = meta_ref[2 * MW + i]
        slot = lax.broadcasted_iota(jnp.int32, (1, BLK), 1) + i * BLK
        valid = slot < end
        tok_raw = tok_ref[0, 0, :].reshape(1, BLK)
        tok = jnp.where(valid, tok_raw, -1)
        ws = jnp.where(valid, ws_ref[0, 0, :].reshape(1, BLK), 0.0)
        tok_c = tok.reshape(BLK, 1)
        ti = lax.broadcasted_iota(jnp.int32, (BLK, T), 1)
        oh = (ti == tok_c).astype(jnp.bfloat16)
        xg = lax.dot_general(oh, x_ref[...], (((1,), (0,)), ((), ())),
                             preferred_element_type=jnp.float32)
        h = lax.dot_general(xg.astype(jnp.bfloat16), w1_ref[0],
                            (((1,), (0,)), ((), ())),
                            preferred_element_type=jnp.float32)
        h = h + b1_ref[0, 0, :][None, :]
        g = 0.5 * h * (1.0 + lax.erf(h * 0.7071067811865476))
        y = lax.dot_general(g.astype(jnp.bfloat16), w2_ref[0],
                            (((1,), (0,)), ((), ())),
                            preferred_element_type=jnp.float32)
        y = y + b2_ref[0, 0, :][None, :]
        wy = (y * ws.reshape(BLK, 1)).astype(jnp.bfloat16)
        tj = lax.broadcasted_iota(jnp.int32, (T, BLK), 0)
        oht = (tj == tok.reshape(1, BLK)).astype(jnp.bfloat16)
        out_ref[...] += lax.dot_general(oht, wy, (((1,), (0,)), ((), ())),
                                        preferred_element_type=jnp.float32)


def _router(xt, gate_w):
    return pl.pallas_call(
        _router_kernel,
        out_shape=[
            jax.ShapeDtypeStruct((T, 128), jnp.float32),
            jax.ShapeDtypeStruct((1, 3 * MW), jnp.int32),
            jax.ShapeDtypeStruct((1, 1), jnp.float32),
        ],
    )(xt, gate_w)


def _ffn(meta, x16, w1_16, b1r, w2_16, b2r, tok_s, ws_s):
    grid_spec = pltpu.PrefetchScalarGridSpec(
        num_scalar_prefetch=1,
        grid=(NT,),
        in_specs=[
            pl.BlockSpec((T, D), lambda i, m: (0, 0)),
            pl.BlockSpec((1, D, F), lambda i, m: (m[i], 0, 0)),
            pl.BlockSpec((1, 1, F), lambda i, m: (m[i], 0, 0)),
            pl.BlockSpec((1, F, D), lambda i, m: (m[i], 0, 0)),
            pl.BlockSpec((1, 1, D), lambda i, m: (m[i], 0, 0)),
            pl.BlockSpec((1, 1, BLK), lambda i, m: (i, 0, 0)),
            pl.BlockSpec((1, 1, BLK), lambda i, m: (i, 0, 0)),
        ],
        out_specs=pl.BlockSpec((T, D), lambda i, m: (0, 0)),
    )
    return pl.pallas_call(
        _ffn_kernel,
        grid_spec=grid_spec,
        out_shape=jax.ShapeDtypeStruct((T, D), jnp.float32),
    )(meta, x16, w1_16, b1r, w2_16, b2r, tok_s, ws_s)


def kernel(x, gate_w, w1, b1, w2, b2):
    orig_shape = x.shape
    xt = x.reshape(T, D)
    dpack, meta, loss = _router(xt, gate_w)
    dest_f = dpack[:, :2].astype(jnp.int32).reshape(NP)
    w_f = dpack[:, 2:4].reshape(NP)
    tok_sorted, w_sorted = _make_sc_dispatch()(dest_f, w_f)
    tok_s = tok_sorted.reshape(NT, 1, BLK)
    ws_s = w_sorted.reshape(NT, 1, BLK)
    out = _ffn(meta.reshape(3 * MW), xt.astype(jnp.bfloat16),
               w1.astype(jnp.bfloat16), b1.reshape(E, 1, F),
               w2.astype(jnp.bfloat16), b2.reshape(E, 1, D), tok_s, ws_s)
    return out.reshape(orig_shape), loss.reshape(())


# speedup vs baseline: 3.3191x; 1.0910x over previous
import functools

import jax
import jax.numpy as jnp
from jax import lax
from jax.experimental import pallas as pl
from jax.experimental.pallas import tpu as pltpu
from jax.experimental.pallas import tpu_sc as plsc

T = 2048
D = 768
F = 3072
E = 8
K = 2
BLK = 256
NP = K * T
P_MAX = 6144
NT = P_MAX // BLK
NW = 32
NPW = NP // NW
MW = 2 * NT


def _router_kernel(x_ref, gw_ref, dpack_ref, meta_ref, loss_ref):
    x = x_ref[...]
    gw = gw_ref[...]
    logits = lax.dot_general(x, gw, (((1,), (1,)), ((), ())),
                             preferred_element_type=jnp.float32)
    m = jnp.max(logits, axis=1, keepdims=True)
    ex = jnp.exp(logits - m)
    probs = ex / jnp.sum(ex, axis=1, keepdims=True)

    mean_p = jnp.mean(probs, axis=0, keepdims=True)
    loss_ref[...] = (float(E) * jnp.sum(mean_p * mean_p)).reshape(1, 1)

    iota_e = lax.broadcasted_iota(jnp.int32, (T, E), 1).astype(jnp.float32)
    m1 = jnp.max(probs, axis=1, keepdims=True)
    i1 = jnp.min(jnp.where(probs >= m1, iota_e, float(E)), axis=1,
                 keepdims=True)
    h0 = (iota_e == i1).astype(jnp.float32)
    probs_m = jnp.where(iota_e == i1, -1.0, probs)
    m2 = jnp.max(probs_m, axis=1, keepdims=True)
    i2 = jnp.min(jnp.where(probs_m >= m2, iota_e, float(E)), axis=1,
                 keepdims=True)
    h1 = (iota_e == i2).astype(jnp.float32)

    denom = m1 + m2
    w0 = m1 / denom
    w1_ = m2 / denom

    ct = h0 + h1
    s = 1
    while s < T:
        shifted = jnp.concatenate(
            [jnp.zeros((s, E), jnp.float32), ct[: T - s, :]], axis=0)
        ct = ct + shifted
        s *= 2
    cnt = ct[T - 1:T, :]
    pc = jnp.ceil(cnt * (1.0 / BLK)) * float(BLK)

    bi = lax.broadcasted_iota(jnp.int32, (E, E), 0).astype(jnp.float32)
    ei = lax.broadcasted_iota(jnp.int32, (E, E), 1).astype(jnp.float32)
    u_excl = (bi < ei).astype(jnp.float32)
    u_incl = (bi >= ei).astype(jnp.float32)
    ident = (bi == ei).astype(jnp.float32)
    bo = lax.dot_general(pc, u_excl, (((1,), (0,)), ((), ())),
                         preferred_element_type=jnp.float32,
                         precision=lax.Precision.HIGHEST)
    boincl_col = lax.dot_general(u_incl, pc, (((1,), (1,)), ((), ())),
                                 preferred_element_type=jnp.float32,
                                 precision=lax.Precision.HIGHEST)
    pc_col = lax.dot_general(ident, pc, (((1,), (1,)), ((), ())),
                             preferred_element_type=jnp.float32,
                             precision=lax.Precision.HIGHEST)
    cnt_col = lax.dot_general(ident, cnt, (((1,), (1,)), ((), ())),
                              preferred_element_type=jnp.float32,
                              precision=lax.Precision.HIGHEST)
    bo_col = boincl_col - pc_col

    d0 = jnp.sum(h0 * (bo + ct - 1.0), axis=1, keepdims=True)
    d1 = jnp.sum(h1 * (bo + ct - 1.0), axis=1, keepdims=True)
    li = lax.broadcasted_iota(jnp.int32, (T, 128), 1)
    dpack_ref[...] = (jnp.where(li == 0, d0, 0.0) +
                      jnp.where(li == 1, d1, 0.0) +
                      jnp.where(li == 2, w0, 0.0) +
                      jnp.where(li == 3, w1_, 0.0))

    ti = lax.broadcasted_iota(jnp.int32, (E, MW), 1).astype(
        jnp.float32) * float(BLK)
    raw = jnp.sum((ti >= boincl_col).astype(jnp.float32), axis=0,
                  keepdims=True)
    e_iota_row = lax.broadcasted_iota(jnp.int32, (1, E), 1).astype(jnp.float32)
    me = jnp.max(jnp.where(cnt > 0.0, e_iota_row, -1.0), axis=1,
                 keepdims=True)
    e_of_tile = jnp.minimum(raw, me)
    nt_used = jnp.sum(pc) * (1.0 / BLK)
    e_iota_col = lax.broadcasted_iota(jnp.int32, (E, 1), 0).astype(jnp.float32)
    tile_oh = (e_of_tile == e_iota_col).astype(jnp.float32)
    end_of_tile = jnp.sum(tile_oh * (bo_col + cnt_col), axis=0,
                          keepdims=True)
    meta = jnp.concatenate(
        [e_of_tile,
         jnp.zeros((1, MW), jnp.float32) + nt_used,
         end_of_tile], axis=1).astype(jnp.int32)
    meta_ref[...] = meta


PPS = NP // 16
HALF = P_MAX // 2
CHO = HALF // 16


@functools.lru_cache(maxsize=1)
def _make_sc_dispatch():
    mesh = plsc.VectorSubcoreMesh(core_axis_name="c", subcore_axis_name="s")

    @functools.partial(
        pl.kernel,
        mesh=mesh,
        out_type=[
            jax.ShapeDtypeStruct((P_MAX,), jnp.int32),
            jax.ShapeDtypeStruct((P_MAX,), jnp.float32),
        ],
        scratch_types=[
            pltpu.VMEM((PPS,), jnp.int32),
            pltpu.VMEM((PPS,), jnp.int32),
            pltpu.VMEM((PPS,), jnp.int32),
            pltpu.VMEM((PPS,), jnp.float32),
            pltpu.VMEM_SHARED((HALF + 8,), jnp.int32),
            pltpu.VMEM_SHARED((HALF + 8,), jnp.float32),
        ],
    )
    def sc_dispatch(dest_hbm, w_hbm, tok_out, w_out, idx_v, lidx_v, tokv,
                    wv, stok, sw):
        c = lax.axis_index("c")
        s = lax.axis_index("s")
        pb = s * PPS
        half = c * HALF
        pltpu.sync_copy(dest_hbm.at[pl.ds(pb, PPS)], idx_v)
        pltpu.sync_copy(w_hbm.at[pl.ds(pb, PPS)], wv)
        for j in range(PPS // 16):
            d = idx_v[pl.ds(j * 16, 16)]
            loc = d - half
            ok = (loc >= 0) & (loc < HALF)
            lidx_v[pl.ds(j * 16, 16)] = jnp.where(ok, loc, HALF)
            p = lax.iota(jnp.int32, 16) + (pb + j * 16)
            tokv[pl.ds(j * 16, 16)] = lax.shift_right_logical(p, 1)
        pltpu.sync_copy(tokv, stok.at[lidx_v])
        pltpu.sync_copy(wv, sw.at[lidx_v])
        plsc.subcore_barrier()
        pltpu.sync_copy(stok.at[pl.ds(s * CHO, CHO)],
                        tokv.at[pl.ds(0, CHO)])
        pltpu.sync_copy(sw.at[pl.ds(s * CHO, CHO)], wv.at[pl.ds(0, CHO)])
        pltpu.sync_copy(tokv.at[pl.ds(0, CHO)],
                        tok_out.at[pl.ds(half + s * CHO, CHO)])
        pltpu.sync_copy(wv.at[pl.ds(0, CHO)],
                        w_out.at[pl.ds(half + s * CHO, CHO)])

    return sc_dispatch


def _ffn_kernel(meta_ref, x_ref, w1_ref, b1_ref, w2_ref, b2_ref, tok_ref,
                ws_ref, out_ref):
    i = pl.program_id(0)
    nt_used = meta_ref[MW]

    @pl.when(i == 0)
    def _init():
        out_ref[...] = jnp.zeros((T, D), jnp.float32)

    @pl.when(i < nt_used)
    def _body():
        end = meta_ref[2 * MW + i]
        slot = lax.broadcasted_iota(jnp.int32, (1, BLK), 1) + i * BLK
        valid = slot < end
        tok_raw = tok_ref[0, 0, :].reshape(1, BLK)
        tok = jnp.where(valid, tok_raw, -1)
        ws = jnp.where(valid, ws_ref[0, 0, :].reshape(1, BLK), 0.0)
        tok_c = tok.reshape(BLK, 1)
        ti = lax.broadcasted_iota(jnp.int32, (BLK, T), 1)
        oh = (ti == tok_c).astype(jnp.bfloat16)
        xg = lax.dot_general(oh, x_ref[...], (((1,), (0,)), ((), ())),
                             preferred_element_type=jnp.float32)
        h = lax.dot_general(xg.astype(jnp.bfloat16), w1_ref[0],
                            (((1,), (0,)), ((), ())),
                            preferred_element_type=jnp.float32)
        h = h + b1_ref[0, 0, :][None, :]
        g = 0.5 * h * (1.0 + lax.erf(h * 0.7071067811865476))
        y = lax.dot_general(g.astype(jnp.bfloat16), w2_ref[0],
                            (((1,), (0,)), ((), ())),
                            preferred_element_type=jnp.float32)
        y = y + b2_ref[0, 0, :][None, :]
        wy = (y * ws.reshape(BLK, 1)).astype(jnp.bfloat16)
        tj = lax.broadcasted_iota(jnp.int32, (T, BLK), 0)
        oht = (tj == tok.reshape(1, BLK)).astype(jnp.bfloat16)
        out_ref[...] += lax.dot_general(oht, wy, (((1,), (0,)), ((), ())),
                                        preferred_element_type=jnp.float32)


def _router(xt, gate_w):
    return pl.pallas_call(
        _router_kernel,
        out_shape=[
            jax.ShapeDtypeStruct((T, 128), jnp.float32),
            jax.ShapeDtypeStruct((1, 3 * MW), jnp.int32),
            jax.ShapeDtypeStruct((1, 1), jnp.float32),
        ],
    )(xt, gate_w)


def _ffn(meta, x16, w1_16, b1r, w2_16, b2r, tok_s, ws_s):
    grid_spec = pltpu.PrefetchScalarGridSpec(
        num_scalar_prefetch=1,
        grid=(NT,),
        in_specs=[
            pl.BlockSpec((T, D), lambda i, m: (0, 0)),
            pl.BlockSpec((1, D, F), lambda i, m: (m[i], 0, 0)),
            pl.BlockSpec((1, 1, F), lambda i, m: (m[i], 0, 0)),
            pl.BlockSpec((1, F, D), lambda i, m: (m[i], 0, 0)),
            pl.BlockSpec((1, 1, D), lambda i, m: (m[i], 0, 0)),
            pl.BlockSpec((1, 1, BLK), lambda i, m: (i, 0, 0)),
            pl.BlockSpec((1, 1, BLK), lambda i, m: (i, 0, 0)),
        ],
        out_specs=pl.BlockSpec((T, D), lambda i, m: (0, 0)),
    )
    return pl.pallas_call(
        _ffn_kernel,
        grid_spec=grid_spec,
        out_shape=jax.ShapeDtypeStruct((T, D), jnp.float32),
    )(meta, x16, w1_16, b1r, w2_16, b2r, tok_s, ws_s)


def kernel(x, gate_w, w1, b1, w2, b2):
    orig_shape = x.shape
    xt = x.reshape(T, D)
    dpack, meta, loss = _router(xt, gate_w)
    dest_f = dpack[:, :2].astype(jnp.int32).reshape(NP)
    w_f = dpack[:, 2:4].reshape(NP)
    tok_sorted, w_sorted = _make_sc_dispatch()(dest_f, w_f)
    tok_s = tok_sorted.reshape(NT, 1, BLK)
    ws_s = w_sorted.reshape(NT, 1, BLK)
    out = _ffn(meta.reshape(3 * MW), xt.astype(jnp.bfloat16),
               w1.astype(jnp.bfloat16), b1.reshape(E, 1, F),
               w2.astype(jnp.bfloat16), b2.reshape(E, 1, D), tok_s, ws_s)
    return out.reshape(orig_shape), loss.reshape(())


# speedup vs baseline: 4.5578x; 1.3732x over previous
import functools

import jax
import jax.numpy as jnp
from jax import lax
from jax.experimental import pallas as pl
from jax.experimental.pallas import tpu as pltpu
from jax.experimental.pallas import tpu_sc as plsc

T = 2048
D = 768
F = 3072
E = 8
K = 2
BLK = 256
NP = K * T
P_MAX = 6144
NT = P_MAX // BLK
NW = 32
NPW = NP // NW
MW = 2 * NT


def _router_kernel(x_ref, gw_ref, dpack_ref, meta_ref, loss_ref):
    x = x_ref[...]
    gw = gw_ref[...]
    logits = lax.dot_general(x, gw, (((1,), (1,)), ((), ())),
                             preferred_element_type=jnp.float32)
    m = jnp.max(logits, axis=1, keepdims=True)
    ex = jnp.exp(logits - m)
    probs = ex / jnp.sum(ex, axis=1, keepdims=True)

    mean_p = jnp.mean(probs, axis=0, keepdims=True)
    loss_ref[...] = (float(E) * jnp.sum(mean_p * mean_p)).reshape(1, 1)

    iota_e = lax.broadcasted_iota(jnp.int32, (T, E), 1).astype(jnp.float32)
    m1 = jnp.max(probs, axis=1, keepdims=True)
    i1 = jnp.min(jnp.where(probs >= m1, iota_e, float(E)), axis=1,
                 keepdims=True)
    h0 = (iota_e == i1).astype(jnp.float32)
    probs_m = jnp.where(iota_e == i1, -1.0, probs)
    m2 = jnp.max(probs_m, axis=1, keepdims=True)
    i2 = jnp.min(jnp.where(probs_m >= m2, iota_e, float(E)), axis=1,
                 keepdims=True)
    h1 = (iota_e == i2).astype(jnp.float32)

    denom = m1 + m2
    w0 = m1 / denom
    w1_ = m2 / denom

    ct = h0 + h1
    s = 1
    while s < T:
        shifted = jnp.concatenate(
            [jnp.zeros((s, E), jnp.float32), ct[: T - s, :]], axis=0)
        ct = ct + shifted
        s *= 2
    cnt = ct[T - 1:T, :]
    pc = jnp.ceil(cnt * (1.0 / BLK)) * float(BLK)

    bi = lax.broadcasted_iota(jnp.int32, (E, E), 0).astype(jnp.float32)
    ei = lax.broadcasted_iota(jnp.int32, (E, E), 1).astype(jnp.float32)
    u_excl = (bi < ei).astype(jnp.float32)
    u_incl = (bi >= ei).astype(jnp.float32)
    ident = (bi == ei).astype(jnp.float32)
    bo = lax.dot_general(pc, u_excl, (((1,), (0,)), ((), ())),
                         preferred_element_type=jnp.float32,
                         precision=lax.Precision.HIGHEST)
    boincl_col = lax.dot_general(u_incl, pc, (((1,), (1,)), ((), ())),
                                 preferred_element_type=jnp.float32,
                                 precision=lax.Precision.HIGHEST)
    pc_col = lax.dot_general(ident, pc, (((1,), (1,)), ((), ())),
                             preferred_element_type=jnp.float32,
                             precision=lax.Precision.HIGHEST)
    cnt_col = lax.dot_general(ident, cnt, (((1,), (1,)), ((), ())),
                              preferred_element_type=jnp.float32,
                              precision=lax.Precision.HIGHEST)
    bo_col = boincl_col - pc_col

    d0 = jnp.sum(h0 * (bo + ct - 1.0), axis=1, keepdims=True)
    d1 = jnp.sum(h1 * (bo + ct - 1.0), axis=1, keepdims=True)
    li = lax.broadcasted_iota(jnp.int32, (T, 128), 1)
    dpack_ref[...] = (jnp.where(li == 0, d0, 0.0) +
                      jnp.where(li == 1, d1, 0.0) +
                      jnp.where(li == 2, w0, 0.0) +
                      jnp.where(li == 3, w1_, 0.0))

    ti = lax.broadcasted_iota(jnp.int32, (E, MW), 1).astype(
        jnp.float32) * float(BLK)
    raw = jnp.sum((ti >= boincl_col).astype(jnp.float32), axis=0,
                  keepdims=True)
    e_iota_row = lax.broadcasted_iota(jnp.int32, (1, E), 1).astype(jnp.float32)
    me = jnp.max(jnp.where(cnt > 0.0, e_iota_row, -1.0), axis=1,
                 keepdims=True)
    e_of_tile = jnp.minimum(raw, me)
    nt_used = jnp.sum(pc) * (1.0 / BLK)
    e_iota_col = lax.broadcasted_iota(jnp.int32, (E, 1), 0).astype(jnp.float32)
    tile_oh = (e_of_tile == e_iota_col).astype(jnp.float32)
    end_of_tile = jnp.sum(tile_oh * (bo_col + cnt_col), axis=0,
                          keepdims=True)
    meta = jnp.concatenate(
        [e_of_tile,
         jnp.zeros((1, MW), jnp.float32) + nt_used,
         end_of_tile], axis=1).astype(jnp.int32)
    meta_ref[...] = meta


PPS = NP // 16
HALF = P_MAX // 2
CHO = HALF // 16


@functools.lru_cache(maxsize=1)
def _make_sc_dispatch():
    mesh = plsc.VectorSubcoreMesh(core_axis_name="c", subcore_axis_name="s")

    @functools.partial(
        pl.kernel,
        mesh=mesh,
        out_type=[
            jax.ShapeDtypeStruct((P_MAX,), jnp.int32),
            jax.ShapeDtypeStruct((P_MAX,), jnp.float32),
        ],
        scratch_types=[
            pltpu.VMEM((PPS,), jnp.int32),
            pltpu.VMEM((PPS,), jnp.int32),
            pltpu.VMEM((PPS,), jnp.int32),
            pltpu.VMEM((PPS,), jnp.float32),
            pltpu.VMEM_SHARED((HALF + 8,), jnp.int32),
            pltpu.VMEM_SHARED((HALF + 8,), jnp.float32),
        ],
    )
    def sc_dispatch(dest_hbm, w_hbm, tok_out, w_out, idx_v, lidx_v, tokv,
                    wv, stok, sw):
        c = lax.axis_index("c")
        s = lax.axis_index("s")
        pb = s * PPS
        half = c * HALF
        pltpu.sync_copy(dest_hbm.at[pl.ds(pb, PPS)], idx_v)
        pltpu.sync_copy(w_hbm.at[pl.ds(pb, PPS)], wv)
        for j in range(PPS // 16):
            d = idx_v[pl.ds(j * 16, 16)]
            loc = d - half
            ok = (loc >= 0) & (loc < HALF)
            lidx_v[pl.ds(j * 16, 16)] = jnp.where(ok, loc, HALF)
            p = lax.iota(jnp.int32, 16) + (pb + j * 16)
            tokv[pl.ds(j * 16, 16)] = lax.shift_right_logical(p, 1)
        pltpu.sync_copy(tokv, stok.at[lidx_v])
        pltpu.sync_copy(wv, sw.at[lidx_v])
        plsc.subcore_barrier()
        pltpu.sync_copy(stok.at[pl.ds(s * CHO, CHO)],
                        tokv.at[pl.ds(0, CHO)])
        pltpu.sync_copy(sw.at[pl.ds(s * CHO, CHO)], wv.at[pl.ds(0, CHO)])
        pltpu.sync_copy(tokv.at[pl.ds(0, CHO)],
                        tok_out.at[pl.ds(half + s * CHO, CHO)])
        pltpu.sync_copy(wv.at[pl.ds(0, CHO)],
                        w_out.at[pl.ds(half + s * CHO, CHO)])

    return sc_dispatch


def _ffn_kernel(meta_ref, x_ref, w1_ref, b1_ref, w2_ref, b2_ref, tok_ref,
                ws_ref, out_ref):
    i = pl.program_id(0)
    nt_used = meta_ref[MW]

    @pl.when(i == 0)
    def _init():
        out_ref[...] = jnp.zeros((T, D), jnp.float32)

    @pl.when(i < nt_used)
    def _body():
        end = meta_ref[2 * MW + i]
        slot = lax.broadcasted_iota(jnp.int32, (1, BLK), 1) + i * BLK
        valid = slot < end
        tok_raw = tok_ref[0, 0, :].reshape(1, BLK)
        tok = jnp.where(valid, tok_raw, -1)
        ws = jnp.where(valid, ws_ref[0, 0, :].reshape(1, BLK), 0.0)
        tok_c = tok.reshape(BLK, 1)
        ti = lax.broadcasted_iota(jnp.int32, (BLK, T), 1)
        oh = (ti == tok_c).astype(jnp.bfloat16)
        xg = lax.dot_general(oh, x_ref[...], (((1,), (0,)), ((), ())),
                             preferred_element_type=jnp.float32)
        h = lax.dot_general(xg, w1_ref[0], (((1,), (0,)), ((), ())),
                            preferred_element_type=jnp.float32)
        h = h + b1_ref[0, 0, :][None, :]
        g = 0.5 * h * (1.0 + lax.erf(h * 0.7071067811865476))
        y = lax.dot_general(g, w2_ref[0], (((1,), (0,)), ((), ())),
                            preferred_element_type=jnp.float32)
        y = y + b2_ref[0, 0, :][None, :]
        wy = (y * ws.reshape(BLK, 1)).astype(jnp.bfloat16)
        tj = lax.broadcasted_iota(jnp.int32, (T, BLK), 0)
        oht = (tj == tok.reshape(1, BLK)).astype(jnp.bfloat16)
        out_ref[...] += lax.dot_general(oht, wy, (((1,), (0,)), ((), ())),
                                        preferred_element_type=jnp.float32)


def _router(xt, gate_w):
    return pl.pallas_call(
        _router_kernel,
        out_shape=[
            jax.ShapeDtypeStruct((T, 128), jnp.float32),
            jax.ShapeDtypeStruct((1, 3 * MW), jnp.int32),
            jax.ShapeDtypeStruct((1, 1), jnp.float32),
        ],
    )(xt, gate_w)


def _ffn(meta, x16, w1_16, b1r, w2_16, b2r, tok_s, ws_s):
    grid_spec = pltpu.PrefetchScalarGridSpec(
        num_scalar_prefetch=1,
        grid=(NT,),
        in_specs=[
            pl.BlockSpec((T, D), lambda i, m: (0, 0)),
            pl.BlockSpec((1, D, F), lambda i, m: (m[i], 0, 0)),
            pl.BlockSpec((1, 1, F), lambda i, m: (m[i], 0, 0)),
            pl.BlockSpec((1, F, D), lambda i, m: (m[i], 0, 0)),
            pl.BlockSpec((1, 1, D), lambda i, m: (m[i], 0, 0)),
            pl.BlockSpec((1, 1, BLK), lambda i, m: (i, 0, 0)),
            pl.BlockSpec((1, 1, BLK), lambda i, m: (i, 0, 0)),
        ],
        out_specs=pl.BlockSpec((T, D), lambda i, m: (0, 0)),
    )
    return pl.pallas_call(
        _ffn_kernel,
        grid_spec=grid_spec,
        out_shape=jax.ShapeDtypeStruct((T, D), jnp.float32),
    )(meta, x16, w1_16, b1r, w2_16, b2r, tok_s, ws_s)


def kernel(x, gate_w, w1, b1, w2, b2):
    orig_shape = x.shape
    xt = x.reshape(T, D)
    dpack, meta, loss = _router(xt, gate_w)
    dest_f = dpack[:, :2].astype(jnp.int32).reshape(NP)
    w_f = dpack[:, 2:4].reshape(NP)
    tok_sorted, w_sorted = _make_sc_dispatch()(dest_f, w_f)
    tok_s = tok_sorted.reshape(NT, 1, BLK)
    ws_s = w_sorted.reshape(NT, 1, BLK)
    out = _ffn(meta.reshape(3 * MW), xt.astype(jnp.bfloat16),
               w1, b1.reshape(E, 1, F),
               w2, b2.reshape(E, 1, D), tok_s, ws_s)
    return out.reshape(orig_shape), loss.reshape(())
